# Initial kernel scaffold; baseline (speedup 1.0000x reference)
import jax
import jax.numpy as jnp
from jax.experimental import pallas as pl


def kernel(x, Wqk, bqk, Wv, bv, Wo, bo, rotations):
    raise NotImplementedError("write your pallas kernel here")


# speedup vs baseline: 4.4716x; 4.4716x over previous
import numpy as np
import jax
import jax.numpy as jnp
from jax.experimental import pallas as pl

B, S, D, H = 2, 8192, 1024, 16
HD = D // H
NB, NR, CHUNK = 64, 4, 64
SCALE = float(np.sqrt(HD))


def _proj_kernel(x_ref, wqk_ref, bqk_ref, wv_ref, bv_ref, qk_ref, v_ref):
    x = x_ref[...]
    qk_ref[...] = jnp.dot(x, wqk_ref[...], preferred_element_type=jnp.float32) + bqk_ref[...]
    v_ref[...] = jnp.dot(x, wv_ref[...], preferred_element_type=jnp.float32) + bv_ref[...]


def _out_proj_kernel(a_ref, w_ref, b_ref, o_ref):
    o_ref[...] = jnp.dot(a_ref[...], w_ref[...], preferred_element_type=jnp.float32) + b_ref[...]


QB = 128
KB = 256


def _attn_kernel(qk_ref, v_ref, bktr_ref, bktc_ref, o_ref):
    def body(c, carry):
        qs = c * QB
        ks = jnp.maximum(c - 1, 0) * QB
        q = qk_ref[0, pl.ds(qs, QB), :]
        k = qk_ref[0, pl.ds(ks, KB), :]
        vv = v_ref[0, pl.ds(ks, KB), :]
        bq = bktc_ref[0, pl.ds(qs, QB), :]
        bk = bktr_ref[0, :, pl.ds(ks, KB)]
        s = jax.lax.dot_general(q, k, (((1,), (1,)), ((), ())),
                                preferred_element_type=jnp.float32) / SCALE
        rowi = jax.lax.broadcasted_iota(jnp.int32, (QB, KB), 0)
        coli = jax.lax.broadcasted_iota(jnp.int32, (QB, KB), 1)
        q_abs = qs + rowi
        k_abs = ks + coli
        mask = (bq != bk) | (k_abs > q_abs) | (k_abs < (q_abs // CHUNK) * CHUNK - CHUNK)
        s = jnp.where(mask, -1e30, s)
        m = jnp.max(s, axis=1, keepdims=True)
        e = jnp.where(mask, 0.0, jnp.exp(s - m))
        den = jnp.sum(e, axis=1, keepdims=True)
        o = jnp.dot(e, vv, preferred_element_type=jnp.float32) / den
        o_ref[0, pl.ds(qs, QB), :] = o
        return carry

    jax.lax.fori_loop(0, S // QB, body, 0)


def kernel(x, Wqk, bqk, Wv, bv, Wo, bo, rotations):
    b, s, d = x.shape
    xf = x.reshape(b * s, d)
    TS = 512

    qkf, vf = pl.pallas_call(
        _proj_kernel,
        grid=(b * s // TS,),
        in_specs=[
            pl.BlockSpec((TS, d), lambda i: (i, 0)),
            pl.BlockSpec((d, d), lambda i: (0, 0)),
            pl.BlockSpec((1, d), lambda i: (0, 0)),
            pl.BlockSpec((d, d), lambda i: (0, 0)),
            pl.BlockSpec((1, d), lambda i: (0, 0)),
        ],
        out_specs=[
            pl.BlockSpec((TS, d), lambda i: (i, 0)),
            pl.BlockSpec((TS, d), lambda i: (i, 0)),
        ],
        out_shape=[
            jax.ShapeDtypeStruct((b * s, d), jnp.float32),
            jax.ShapeDtypeStruct((b * s, d), jnp.float32),
        ],
    )(xf, Wqk.T, bqk.reshape(1, d), Wv.T, bv.reshape(1, d))

    qk_hash = (x @ Wqk.T + bqk).reshape(b, s, H, HD).transpose(0, 2, 1, 3)
    pos = jnp.arange(s)[None, None, :]
    sort_idx_l, bs_l = [], []
    for r in range(NR):
        rotated = jnp.einsum('bhld,hdk->bhlk', qk_hash, rotations[r])
        rotated = jnp.concatenate([rotated, -rotated], axis=-1)
        buckets = jnp.argmax(rotated, axis=-1)
        sort_keys = buckets * s + pos
        sort_idx = jnp.argsort(sort_keys, axis=-1)
        sort_idx_l.append(sort_idx)
        bs_l.append(jnp.take_along_axis(buckets, sort_idx, axis=-1))
    sort_idx_all = jnp.stack(sort_idx_l)
    b_s = jnp.stack(bs_l).astype(jnp.int32)

    qk4 = qkf.reshape(b, s, H, HD).transpose(0, 2, 1, 3)
    v4 = vf.reshape(b, s, H, HD).transpose(0, 2, 1, 3)
    qk_s = jnp.take_along_axis(qk4[None], sort_idx_all[..., None], axis=3)
    v_s = jnp.take_along_axis(v4[None], sort_idx_all[..., None], axis=3)

    G = NR * b * H
    qk_s = qk_s.reshape(G, s, HD)
    v_s = v_s.reshape(G, s, HD)
    bkt_row = b_s.reshape(G, 1, s)
    bkt_col = b_s.reshape(G, s, 1)

    o = pl.pallas_call(
        _attn_kernel,
        grid=(G,),
        in_specs=[
            pl.BlockSpec((1, s, HD), lambda i: (i, 0, 0)),
            pl.BlockSpec((1, s, HD), lambda i: (i, 0, 0)),
            pl.BlockSpec((1, 1, s), lambda i: (i, 0, 0)),
            pl.BlockSpec((1, s, 1), lambda i: (i, 0, 0)),
        ],
        out_specs=pl.BlockSpec((1, s, HD), lambda i: (i, 0, 0)),
        out_shape=jax.ShapeDtypeStruct((G, s, HD), jnp.float32),
    )(qk_s, v_s, bkt_row, bkt_col)

    o = o.reshape(NR, b, H, s, HD)
    inv = jnp.argsort(sort_idx_all, axis=-1)
    o_un = jnp.take_along_axis(o, inv[..., None], axis=3)
    acc = o_un.sum(axis=0) / NR
    a = acc.transpose(0, 2, 1, 3).reshape(b * s, d)

    out = pl.pallas_call(
        _out_proj_kernel,
        grid=(b * s // TS,),
        in_specs=[
            pl.BlockSpec((TS, d), lambda i: (i, 0)),
            pl.BlockSpec((d, d), lambda i: (0, 0)),
            pl.BlockSpec((1, d), lambda i: (0, 0)),
        ],
        out_specs=pl.BlockSpec((TS, d), lambda i: (i, 0)),
        out_shape=jax.ShapeDtypeStruct((b * s, d), jnp.float32),
    )(a, Wo.T, bo.reshape(1, d))
    return out.reshape(b, s, d)
